# baseline (speedup 1.0000x reference)
import functools

import jax
import jax.numpy as jnp
from jax import lax
from jax.experimental import pallas as pl
from jax.experimental.pallas import tpu as pltpu
from jax.experimental.pallas import tpu_sc as plsc

N_NODES = 9992
D = 32
W = 128
B = 16384

NC = 2
NS = 16
NW = NC * NS
BPW = B // NW
L = 16
IDX_CH = 128
NCH = BPW // IDX_CH


def _tc_norm_body(emb_ref, gamma_ref, beta_ref, out_ref):
    x = emb_ref[...]
    sq = jnp.sum(x * x, axis=1, keepdims=True)
    norm = jnp.sqrt(sq)
    scale = jnp.minimum(1.0, 1.0 / jnp.maximum(norm, 1e-7))
    x = x * scale
    mean = jnp.mean(x, axis=0, keepdims=True)
    var = jnp.mean((x - mean) * (x - mean), axis=0, keepdims=True)
    a = gamma_ref[...] / jnp.sqrt(var + 1e-5)
    out_ref[:, pl.ds(0, D)] = (x - mean) * a + beta_ref[...]
    out_ref[:, pl.ds(D, W - D)] = jnp.zeros((N_NODES, W - D), jnp.float32)


def _normalize_table(emb_head, bn_gamma, bn_beta):
    return pl.pallas_call(
        _tc_norm_body,
        out_shape=jax.ShapeDtypeStruct((N_NODES, W), jnp.float32),
    )(emb_head, bn_gamma.reshape(1, D), bn_beta.reshape(1, D))


def _sc_scores_body(embs_hbm, rel_hbm, hidx_hbm, ridx_hbm, tidx_hbm, out_hbm,
                    hidx_v, ridx_v, tidx_v, hrows, rrows, trows, scores_v,
                    sem0, sem1):
    wid = lax.axis_index("s") * NC + lax.axis_index("c")
    base = wid * BPW
    sems = (sem0, sem1)

    pltpu.sync_copy(hidx_hbm.at[pl.ds(wid * NCH, NCH)], hidx_v)
    pltpu.sync_copy(ridx_hbm.at[pl.ds(wid * NCH, NCH)], ridx_v)
    pltpu.sync_copy(tidx_hbm.at[pl.ds(wid * NCH, NCH)], tidx_v)

    def fire(j):
        buf = j % 2
        sem = sems[buf]
        bslice = pl.ds(buf * IDX_CH, IDX_CH)
        return [
            pltpu.async_copy(embs_hbm.at[hidx_v.at[j]], hrows.at[bslice], sem),
            pltpu.async_copy(rel_hbm.at[ridx_v.at[j]], rrows.at[bslice], sem),
            pltpu.async_copy(embs_hbm.at[tidx_v.at[j]], trows.at[bslice], sem),
        ]

    lanes = lax.iota(jnp.int32, L)
    inflight = fire(0)
    for j in range(NCH):
        for c in inflight:
            c.wait()
        inflight = fire(j + 1) if j + 1 < NCH else []
        buf0 = (j % 2) * IDX_CH

        def group_body(g, carry):
            e0 = buf0 + g * L
            acc = jnp.zeros((L,), jnp.float32)
            for k in range(L):
                e = e0 + k
                v = (hrows[e, pl.ds(0, L)] * rrows[e, pl.ds(0, L)]
                     * trows[e, pl.ds(0, L)])
                v += (hrows[e, pl.ds(L, L)] * rrows[e, pl.ds(L, L)]
                      * trows[e, pl.ds(L, L)])
                s = jnp.sum(v)
                acc = jnp.where(lanes == k, s, acc)
            scores_v[pl.ds(j * IDX_CH + g * L, L)] = acc
            return carry

        lax.fori_loop(0, IDX_CH // L, group_body, 0)

    pltpu.sync_copy(scores_v, out_hbm.at[pl.ds(base, BPW)])


def _sc_scores(embs, rel_pad, hidx, ridx, tidx):
    mesh = plsc.VectorSubcoreMesh(core_axis_name="c", subcore_axis_name="s")
    kern = functools.partial(
        pl.kernel,
        out_type=jax.ShapeDtypeStruct((B,), jnp.float32),
        mesh=mesh,
        compiler_params=pltpu.CompilerParams(needs_layout_passes=False),
        scratch_types=[
            pltpu.VMEM((NCH, IDX_CH), jnp.int32),
            pltpu.VMEM((NCH, IDX_CH), jnp.int32),
            pltpu.VMEM((NCH, IDX_CH), jnp.int32),
            pltpu.VMEM((2 * IDX_CH, W), jnp.float32),
            pltpu.VMEM((2 * IDX_CH, W), jnp.float32),
            pltpu.VMEM((2 * IDX_CH, W), jnp.float32),
            pltpu.VMEM((BPW,), jnp.float32),
            pltpu.SemaphoreType.DMA,
            pltpu.SemaphoreType.DMA,
        ],
    )(_sc_scores_body)
    return kern(embs, rel_pad, hidx, ridx, tidx)


def _tc_loss_body(s_ref, preds_ref, loss_ref):
    s = s_ref[...]
    preds_ref[...] = jax.nn.sigmoid(s)
    sp = jnp.maximum(-s, 0.0) + jnp.log1p(jnp.exp(-jnp.abs(s)))
    loss_ref[...] = jnp.mean(sp).reshape(1, 1)


def _preds_loss(scores):
    s2d = scores.reshape(B // 128, 128)
    preds2d, loss2d = pl.pallas_call(
        _tc_loss_body,
        out_shape=(
            jax.ShapeDtypeStruct((B // 128, 128), jnp.float32),
            jax.ShapeDtypeStruct((1, 1), jnp.float32),
        ),
    )(s2d)
    return preds2d.reshape(B), loss2d[0, 0]


def kernel(pos_edges, emb_table, bn_gamma, bn_beta, rel_table):
    embs = _normalize_table(emb_table[:N_NODES], bn_gamma, bn_beta)
    rel_pad = jnp.pad(rel_table, ((0, 0), (0, W - D)))
    hidx = pos_edges[:, 0].astype(jnp.int32).reshape(NW * NCH, IDX_CH)
    ridx = pos_edges[:, 1].astype(jnp.int32).reshape(NW * NCH, IDX_CH)
    tidx = pos_edges[:, 2].astype(jnp.int32).reshape(NW * NCH, IDX_CH)
    scores = _sc_scores(embs, rel_pad, hidx, ridx, tidx)
    preds, loss = _preds_loss(scores)
    return (loss, preds)


# speedup vs baseline: 1.2491x; 1.2491x over previous
import functools

import jax
import jax.numpy as jnp
from jax import lax
from jax.experimental import pallas as pl
from jax.experimental.pallas import tpu as pltpu
from jax.experimental.pallas import tpu_sc as plsc

N_NODES = 9992
D = 32
B = 16384

NC = 2
NS = 16
NW = NC * NS
BPW = B // NW
L = 16


RP = N_NODES * D // 128


def _tc_norm_body(emb_ref, gamma_ref, beta_ref, out_ref):
    x = emb_ref[...]
    l4 = lax.broadcasted_iota(jnp.int32, (128, 4), 0)
    c4 = lax.broadcasted_iota(jnp.int32, (128, 4), 1)
    G = (l4 // D == c4).astype(jnp.float32)
    ld = lax.broadcasted_iota(jnp.int32, (128, D), 0)
    cd = lax.broadcasted_iota(jnp.int32, (128, D), 1)
    F = (ld % D == cd).astype(jnp.float32)

    sumsq = jax.lax.dot(x * x, G)
    norm = jnp.sqrt(sumsq)
    scale4 = jnp.minimum(1.0, 1.0 / jnp.maximum(norm, 1e-7))
    xs = x * jax.lax.dot(scale4, G.T)

    s1 = jnp.sum(xs, axis=0, keepdims=True)
    mean = jax.lax.dot(s1, F) / N_NODES
    xc = xs - jax.lax.dot(mean, F.T)
    s2 = jnp.sum(xc * xc, axis=0, keepdims=True)
    var = jax.lax.dot(s2, F) / N_NODES
    a = gamma_ref[...] / jnp.sqrt(var + 1e-5)
    c = beta_ref[...] - mean * a
    y = xs * jax.lax.dot(a, F.T) + jax.lax.dot(c, F.T)
    out_ref[...] = y.reshape(N_NODES * D)


def _normalize_table(emb_head2, bn_gamma, bn_beta):
    flat = pl.pallas_call(
        _tc_norm_body,
        out_shape=jax.ShapeDtypeStruct((N_NODES * D,), jnp.float32),
    )(emb_head2, bn_gamma.reshape(1, D), bn_beta.reshape(1, D))
    return flat.reshape(N_NODES, D)


def _sc_scores_body(embs_hbm, rel_hbm, hidx_hbm, ridx_hbm, tidx_hbm, out_hbm,
                    hidx_v, ridx_v, tidx_v, hrows, rrows, trows, scores_v,
                    sem):
    wid = lax.axis_index("s") * NC + lax.axis_index("c")
    base = wid * BPW

    pltpu.sync_copy(hidx_hbm.at[wid], hidx_v)
    pltpu.sync_copy(ridx_hbm.at[wid], ridx_v)
    pltpu.sync_copy(tidx_hbm.at[wid], tidx_v)
    copies = [
        pltpu.async_copy(embs_hbm.at[hidx_v], hrows, sem),
        pltpu.async_copy(rel_hbm.at[ridx_v], rrows, sem),
        pltpu.async_copy(embs_hbm.at[tidx_v], trows, sem),
    ]
    for c in copies:
        c.wait()

    lanes = lax.iota(jnp.int32, L)

    def group_body(g, carry):
        e0 = g * L
        acc = jnp.zeros((L,), jnp.float32)
        for k in range(L):
            e = e0 + k
            v = (hrows[e, pl.ds(0, L)] * rrows[e, pl.ds(0, L)]
                 * trows[e, pl.ds(0, L)])
            v += (hrows[e, pl.ds(L, L)] * rrows[e, pl.ds(L, L)]
                  * trows[e, pl.ds(L, L)])
            s = jnp.sum(v)
            acc = jnp.where(lanes == k, s, acc)
        scores_v[pl.ds(e0, L)] = acc
        return carry

    lax.fori_loop(0, BPW // L, group_body, 0)
    pltpu.sync_copy(scores_v, out_hbm.at[pl.ds(base, BPW)])


def _sc_scores(embs, rel_table, hidx, ridx, tidx):
    mesh = plsc.VectorSubcoreMesh(core_axis_name="c", subcore_axis_name="s")
    kern = functools.partial(
        pl.kernel,
        out_type=jax.ShapeDtypeStruct((B,), jnp.float32),
        mesh=mesh,
        compiler_params=pltpu.CompilerParams(
            use_tc_tiling_on_sc=False, needs_layout_passes=False),
        scratch_types=[
            pltpu.VMEM((BPW,), jnp.int32),
            pltpu.VMEM((BPW,), jnp.int32),
            pltpu.VMEM((BPW,), jnp.int32),
            pltpu.VMEM((BPW, D), jnp.float32),
            pltpu.VMEM((BPW, D), jnp.float32),
            pltpu.VMEM((BPW, D), jnp.float32),
            pltpu.VMEM((BPW,), jnp.float32),
            pltpu.SemaphoreType.DMA,
        ],
    )(_sc_scores_body)
    return kern(embs, rel_table, hidx, ridx, tidx)


def _tc_loss_body(s_ref, preds_ref, loss_ref):
    s = s_ref[...]
    preds_ref[...] = jax.nn.sigmoid(s)
    sp = jnp.maximum(-s, 0.0) + jnp.log1p(jnp.exp(-jnp.abs(s)))
    loss_ref[...] = jnp.mean(sp).reshape(1, 1)


def _preds_loss(scores):
    s2d = scores.reshape(B // 128, 128)
    preds2d, loss2d = pl.pallas_call(
        _tc_loss_body,
        out_shape=(
            jax.ShapeDtypeStruct((B // 128, 128), jnp.float32),
            jax.ShapeDtypeStruct((1, 1), jnp.float32),
        ),
    )(s2d)
    return preds2d.reshape(B), loss2d[0, 0]


def kernel(pos_edges, emb_table, bn_gamma, bn_beta, rel_table):
    emb2 = emb_table[:N_NODES].reshape(RP, 128)
    embs = _normalize_table(emb2, bn_gamma, bn_beta)
    hidx = pos_edges[:, 0].astype(jnp.int32).reshape(NW, BPW)
    ridx = pos_edges[:, 1].astype(jnp.int32).reshape(NW, BPW)
    tidx = pos_edges[:, 2].astype(jnp.int32).reshape(NW, BPW)
    scores = _sc_scores(embs, rel_table, hidx, ridx, tidx)
    preds, loss = _preds_loss(scores)
    return (loss, preds)
